# baseline (speedup 1.0000x reference)
import jax
import jax.numpy as jnp
from jax.experimental import pallas as pl

EMB_DIM = 256
DICT_SIZE = 8192
ROW_BLOCK = 256
CHUNK = 832


def _vq_block_kernel(flat_ref, embed_ref, ids_ref, vq_ref):
    z = flat_ref[...]
    emb = embed_ref[...]
    embsq = jnp.sum(emb * emb, axis=0, keepdims=True)
    rowsq = jnp.sum(z * z, axis=1, keepdims=True)
    zbf = (2.0 * z).astype(jnp.bfloat16)
    emb_hi = emb.astype(jnp.bfloat16)
    emb_lo = (emb - emb_hi.astype(jnp.float32)).astype(jnp.bfloat16)
    mm = jnp.dot(zbf, emb_hi, preferred_element_type=jnp.float32)
    dist = (embsq + rowsq) - mm

    m = jnp.full((ROW_BLOCK,), jnp.inf, jnp.float32)
    ids = jnp.zeros((ROW_BLOCK,), jnp.int32)
    bounds = list(range(0, DICT_SIZE, CHUNK)) + [DICT_SIZE]
    for b0, b1 in zip(bounds[:-1], bounds[1:]):
        sub = dist[:, b0:b1]
        a = jnp.argmin(sub, axis=1).astype(jnp.int32) + b0
        v = jnp.min(sub, axis=1)
        take = v < m
        m = jnp.where(take, v.astype(jnp.bfloat16).astype(jnp.float32), m)
        ids = jnp.where(take, a, ids)
    ids_ref[0, 0, :] = ids

    onehot = (jax.lax.broadcasted_iota(jnp.int32, (ROW_BLOCK, DICT_SIZE), 1)
              == ids[:, None]).astype(jnp.bfloat16)
    vq_ref[...] = (
        jax.lax.dot_general(
            onehot, emb_hi,
            dimension_numbers=(((1,), (1,)), ((), ())),
            preferred_element_type=jnp.float32)
        + jax.lax.dot_general(
            onehot, emb_lo,
            dimension_numbers=(((1,), (1,)), ((), ())),
            preferred_element_type=jnp.float32)
    )


def kernel(input, embed):
    B, C, H, W = input.shape
    n = B * H * W
    nblk = n // ROW_BLOCK
    flatten = jnp.transpose(input, (0, 3, 2, 1)).reshape(n, C)
    ids3, vq = pl.pallas_call(
        _vq_block_kernel,
        grid=(nblk,),
        in_specs=[
            pl.BlockSpec((ROW_BLOCK, C), lambda i: (i, 0)),
            pl.BlockSpec((C, DICT_SIZE), lambda i: (0, 0)),
        ],
        out_specs=[
            pl.BlockSpec((1, 1, ROW_BLOCK), lambda i: (i, 0, 0)),
            pl.BlockSpec((ROW_BLOCK, C), lambda i: (i, 0)),
        ],
        out_shape=[
            jax.ShapeDtypeStruct((nblk, 1, ROW_BLOCK), jnp.int32),
            jax.ShapeDtypeStruct((n, C), jnp.float32),
        ],
    )(flatten, embed)
    ids = ids3.reshape(B, H, W)
    vqs = vq.reshape(B, H, W, C)
    out = jnp.transpose(vqs, (0, 3, 2, 1))
    return out, ids


# speedup vs baseline: 1.6316x; 1.6316x over previous
import jax
import jax.numpy as jnp
from jax.experimental import pallas as pl

EMB_DIM = 256
DICT_SIZE = 8192
ROW_BLOCK = 256
CHUNK = 1664


def _vq_block_kernel(flat_ref, embed_ref, ids_ref, vq_ref):
    z = flat_ref[...]
    emb = embed_ref[...]
    embsq = jnp.sum(emb * emb, axis=0, keepdims=True)
    rowsq = jnp.sum(z * z, axis=1, keepdims=True)
    zbf = (2.0 * z).astype(jnp.bfloat16)
    emb_hi = emb.astype(jnp.bfloat16)
    emb_lo = (emb - emb_hi.astype(jnp.float32)).astype(jnp.bfloat16)
    mm = jnp.dot(zbf, emb_hi, preferred_element_type=jnp.float32)
    dist = (embsq + rowsq) - mm

    m = jnp.full((ROW_BLOCK,), jnp.inf, jnp.float32)
    ids = jnp.zeros((ROW_BLOCK,), jnp.int32)
    bounds = list(range(0, DICT_SIZE, CHUNK)) + [DICT_SIZE]
    for b0, b1 in zip(bounds[:-1], bounds[1:]):
        sub = dist[:, b0:b1]
        a = jnp.argmin(sub, axis=1).astype(jnp.int32) + b0
        v = jnp.min(sub, axis=1)
        take = v < m
        m = jnp.where(take, v.astype(jnp.bfloat16).astype(jnp.float32), m)
        ids = jnp.where(take, a, ids)
    ids_ref[0, 0, :] = ids

    onehot = (jax.lax.broadcasted_iota(jnp.int32, (ROW_BLOCK, DICT_SIZE), 1)
              == ids[:, None]).astype(jnp.bfloat16)
    vq_ref[...] = (
        jax.lax.dot_general(
            onehot, emb_hi,
            dimension_numbers=(((1,), (1,)), ((), ())),
            preferred_element_type=jnp.float32)
        + jax.lax.dot_general(
            onehot, emb_lo,
            dimension_numbers=(((1,), (1,)), ((), ())),
            preferred_element_type=jnp.float32)
    )


def kernel(input, embed):
    B, C, H, W = input.shape
    n = B * H * W
    nblk = n // ROW_BLOCK
    flatten = jnp.transpose(input, (0, 3, 2, 1)).reshape(n, C)
    ids3, vq = pl.pallas_call(
        _vq_block_kernel,
        grid=(nblk,),
        in_specs=[
            pl.BlockSpec((ROW_BLOCK, C), lambda i: (i, 0)),
            pl.BlockSpec((C, DICT_SIZE), lambda i: (0, 0)),
        ],
        out_specs=[
            pl.BlockSpec((1, 1, ROW_BLOCK), lambda i: (i, 0, 0)),
            pl.BlockSpec((ROW_BLOCK, C), lambda i: (i, 0)),
        ],
        out_shape=[
            jax.ShapeDtypeStruct((nblk, 1, ROW_BLOCK), jnp.int32),
            jax.ShapeDtypeStruct((n, C), jnp.float32),
        ],
    )(flatten, embed)
    ids = ids3.reshape(B, H, W)
    vqs = vq.reshape(B, H, W, C)
    out = jnp.transpose(vqs, (0, 3, 2, 1))
    return out, ids


# speedup vs baseline: 1.6691x; 1.0230x over previous
import jax
import jax.numpy as jnp
from jax.experimental import pallas as pl
from jax.experimental.pallas import tpu as pltpu

EMB_DIM = 256
DICT_SIZE = 8192
ROW_BLOCK = 512
CHUNK = 1664


def _vq_block_kernel(flat_ref, embed_ref, ids_ref, vq_ref, embsq_ref):
    emb = embed_ref[...]

    @pl.when(pl.program_id(0) == 0)
    def _():
        embsq_ref[...] = jnp.sum(emb * emb, axis=0, keepdims=True)

    z = flat_ref[...]
    embsq = embsq_ref[...]
    rowsq = jnp.sum(z * z, axis=1, keepdims=True)
    zbf = (2.0 * z).astype(jnp.bfloat16)
    emb_hi = emb.astype(jnp.bfloat16)
    emb_lo = (emb - emb_hi.astype(jnp.float32)).astype(jnp.bfloat16)
    mm = jnp.dot(zbf, emb_hi, preferred_element_type=jnp.float32)
    dist = (embsq + rowsq) - mm

    m = jnp.full((ROW_BLOCK,), jnp.inf, jnp.float32)
    ids = jnp.zeros((ROW_BLOCK,), jnp.int32)
    bounds = list(range(0, DICT_SIZE, CHUNK)) + [DICT_SIZE]
    for b0, b1 in zip(bounds[:-1], bounds[1:]):
        sub = dist[:, b0:b1]
        a = jnp.argmin(sub, axis=1).astype(jnp.int32) + b0
        v = jnp.min(sub, axis=1)
        take = v < m
        m = jnp.where(take, v.astype(jnp.bfloat16).astype(jnp.float32), m)
        ids = jnp.where(take, a, ids)
    ids_ref[0, 0, :] = ids

    onehot = (jax.lax.broadcasted_iota(jnp.int32, (ROW_BLOCK, DICT_SIZE), 1)
              == ids[:, None]).astype(jnp.bfloat16)
    vq_ref[...] = (
        jax.lax.dot_general(
            onehot, emb_hi,
            dimension_numbers=(((1,), (1,)), ((), ())),
            preferred_element_type=jnp.float32)
        + jax.lax.dot_general(
            onehot, emb_lo,
            dimension_numbers=(((1,), (1,)), ((), ())),
            preferred_element_type=jnp.float32)
    )


def kernel(input, embed):
    B, C, H, W = input.shape
    n = B * H * W
    nblk = n // ROW_BLOCK
    flatten = jnp.transpose(input, (0, 3, 2, 1)).reshape(n, C)
    ids3, vq = pl.pallas_call(
        _vq_block_kernel,
        grid=(nblk,),
        in_specs=[
            pl.BlockSpec((ROW_BLOCK, C), lambda i: (i, 0)),
            pl.BlockSpec((C, DICT_SIZE), lambda i: (0, 0)),
        ],
        out_specs=[
            pl.BlockSpec((1, 1, ROW_BLOCK), lambda i: (i, 0, 0)),
            pl.BlockSpec((ROW_BLOCK, C), lambda i: (i, 0)),
        ],
        out_shape=[
            jax.ShapeDtypeStruct((nblk, 1, ROW_BLOCK), jnp.int32),
            jax.ShapeDtypeStruct((n, C), jnp.float32),
        ],
        scratch_shapes=[pltpu.VMEM((1, DICT_SIZE), jnp.float32)],
    )(flatten, embed)
    ids = ids3.reshape(B, H, W)
    vqs = vq.reshape(B, H, W, C)
    out = jnp.transpose(vqs, (0, 3, 2, 1))
    return out, ids


# speedup vs baseline: 1.8989x; 1.1377x over previous
import jax
import jax.numpy as jnp
from jax.experimental import pallas as pl
from jax.experimental.pallas import tpu as pltpu

EMB_DIM = 256
DICT_SIZE = 8192
ROW_BLOCK = 512
HALF = 256
CHUNK = 1664


def _vq_half(z, emb_hi, emb_lo, embsq):
    rowsq = jnp.sum(z * z, axis=1, keepdims=True)
    zbf = (2.0 * z).astype(jnp.bfloat16)
    mm = jnp.dot(zbf, emb_hi, preferred_element_type=jnp.float32)
    dist = (embsq + rowsq) - mm

    m = jnp.full((HALF,), jnp.inf, jnp.float32)
    ids = jnp.zeros((HALF,), jnp.int32)
    bounds = list(range(0, DICT_SIZE, CHUNK)) + [DICT_SIZE]
    for b0, b1 in zip(bounds[:-1], bounds[1:]):
        sub = dist[:, b0:b1]
        a = jnp.argmin(sub, axis=1).astype(jnp.int32) + b0
        v = jnp.min(sub, axis=1)
        take = v < m
        m = jnp.where(take, v.astype(jnp.bfloat16).astype(jnp.float32), m)
        ids = jnp.where(take, a, ids)

    onehot = (jax.lax.broadcasted_iota(jnp.int32, (HALF, DICT_SIZE), 1)
              == ids[:, None]).astype(jnp.bfloat16)
    vq = (
        jax.lax.dot_general(
            onehot, emb_hi,
            dimension_numbers=(((1,), (1,)), ((), ())),
            preferred_element_type=jnp.float32)
        + jax.lax.dot_general(
            onehot, emb_lo,
            dimension_numbers=(((1,), (1,)), ((), ())),
            preferred_element_type=jnp.float32)
    )
    return ids, vq


def _vq_block_kernel(flat_ref, embed_ref, ids_ref, vq_ref, embsq_ref):
    emb = embed_ref[...]

    @pl.when(pl.program_id(0) == 0)
    def _():
        embsq_ref[...] = jnp.sum(emb * emb, axis=0, keepdims=True)

    embsq = embsq_ref[...]
    emb_hi = emb.astype(jnp.bfloat16)
    emb_lo = (emb - emb_hi.astype(jnp.float32)).astype(jnp.bfloat16)

    ids0, vq0 = _vq_half(flat_ref[0:HALF, :], emb_hi, emb_lo, embsq)
    ids1, vq1 = _vq_half(flat_ref[HALF:ROW_BLOCK, :], emb_hi, emb_lo, embsq)
    ids_ref[0, 0, 0:HALF] = ids0
    ids_ref[0, 0, HALF:ROW_BLOCK] = ids1
    vq_ref[0:HALF, :] = vq0
    vq_ref[HALF:ROW_BLOCK, :] = vq1


def kernel(input, embed):
    B, C, H, W = input.shape
    n = B * H * W
    nblk = n // ROW_BLOCK
    flatten = jnp.transpose(input, (0, 3, 2, 1)).reshape(n, C)
    ids3, vq = pl.pallas_call(
        _vq_block_kernel,
        grid=(nblk,),
        in_specs=[
            pl.BlockSpec((ROW_BLOCK, C), lambda i: (i, 0)),
            pl.BlockSpec((C, DICT_SIZE), lambda i: (0, 0)),
        ],
        out_specs=[
            pl.BlockSpec((1, 1, ROW_BLOCK), lambda i: (i, 0, 0)),
            pl.BlockSpec((ROW_BLOCK, C), lambda i: (i, 0)),
        ],
        out_shape=[
            jax.ShapeDtypeStruct((nblk, 1, ROW_BLOCK), jnp.int32),
            jax.ShapeDtypeStruct((n, C), jnp.float32),
        ],
        scratch_shapes=[pltpu.VMEM((1, DICT_SIZE), jnp.float32)],
    )(flatten, embed)
    ids = ids3.reshape(B, H, W)
    vqs = vq.reshape(B, H, W, C)
    out = jnp.transpose(vqs, (0, 3, 2, 1))
    return out, ids


# speedup vs baseline: 2.4156x; 1.2721x over previous
import jax
import jax.numpy as jnp
from jax.experimental import pallas as pl
from jax.experimental.pallas import tpu as pltpu

EMB_DIM = 256
DICT_SIZE = 8192
ROW_BLOCK = 512
HALF = 256
CHUNK = 1664


def _vq_half(z, emb_hi, embsq):
    rowsq = jnp.sum(z * z, axis=1, keepdims=True)
    zbf = (2.0 * z).astype(jnp.bfloat16)
    mm = jnp.dot(zbf, emb_hi, preferred_element_type=jnp.float32)
    dist = (embsq + rowsq) - mm

    m = jnp.full((HALF,), jnp.inf, jnp.float32)
    ids = jnp.zeros((HALF,), jnp.int32)
    bounds = list(range(0, DICT_SIZE, CHUNK)) + [DICT_SIZE]
    for b0, b1 in zip(bounds[:-1], bounds[1:]):
        sub = dist[:, b0:b1]
        a = jnp.argmin(sub, axis=1).astype(jnp.int32) + b0
        v = jnp.min(sub, axis=1)
        take = v < m
        m = jnp.where(take, v.astype(jnp.bfloat16).astype(jnp.float32), m)
        ids = jnp.where(take, a, ids)

    onehot = (jax.lax.broadcasted_iota(jnp.int32, (HALF, DICT_SIZE), 1)
              == ids[:, None]).astype(jnp.bfloat16)
    vq = jax.lax.dot_general(
        onehot, emb_hi,
        dimension_numbers=(((1,), (1,)), ((), ())),
        preferred_element_type=jnp.float32)
    return ids, vq


def _vq_block_kernel(flat_ref, embed_ref, ids_ref, vq_ref, embsq_ref):
    emb = embed_ref[...]

    @pl.when(pl.program_id(0) == 0)
    def _():
        embsq_ref[...] = jnp.sum(emb * emb, axis=0, keepdims=True)

    embsq = embsq_ref[...]
    emb_hi = emb.astype(jnp.bfloat16)

    ids0, vq0 = _vq_half(flat_ref[0:HALF, :], emb_hi, embsq)
    ids1, vq1 = _vq_half(flat_ref[HALF:ROW_BLOCK, :], emb_hi, embsq)
    ids_ref[0, 0, 0:HALF] = ids0
    ids_ref[0, 0, HALF:ROW_BLOCK] = ids1
    vq_ref[0:HALF, :] = vq0
    vq_ref[HALF:ROW_BLOCK, :] = vq1


def kernel(input, embed):
    B, C, H, W = input.shape
    n = B * H * W
    nblk = n // ROW_BLOCK
    flatten = jnp.transpose(input, (0, 3, 2, 1)).reshape(n, C)
    ids3, vq = pl.pallas_call(
        _vq_block_kernel,
        grid=(nblk,),
        in_specs=[
            pl.BlockSpec((ROW_BLOCK, C), lambda i: (i, 0)),
            pl.BlockSpec((C, DICT_SIZE), lambda i: (0, 0)),
        ],
        out_specs=[
            pl.BlockSpec((1, 1, ROW_BLOCK), lambda i: (i, 0, 0)),
            pl.BlockSpec((ROW_BLOCK, C), lambda i: (i, 0)),
        ],
        out_shape=[
            jax.ShapeDtypeStruct((nblk, 1, ROW_BLOCK), jnp.int32),
            jax.ShapeDtypeStruct((n, C), jnp.float32),
        ],
        scratch_shapes=[pltpu.VMEM((1, DICT_SIZE), jnp.float32)],
    )(flatten, embed)
    ids = ids3.reshape(B, H, W)
    vqs = vq.reshape(B, H, W, C)
    out = jnp.transpose(vqs, (0, 3, 2, 1))
    return out, ids
